# baseline (speedup 1.0000x reference)
import functools

import jax
import jax.numpy as jnp
from jax import lax
from jax.experimental import pallas as pl
from jax.experimental.pallas import tpu as pltpu
from jax.experimental.pallas import tpu_sc as plsc

N = 10000
EMB = 128
EPS = 1e-5

NC = 2
NS = 16
NW = NC * NS
CHUNK = 128
NBUF = 2
NP = N + 8
ROWS_MAIN = (NP // NS) // 8 * 8
TAIL = NP - NS * ROWS_MAIN


NCH0 = 8
NCH1 = 152


def _phase_sizes(nc):
    sizes = [48] * (nc // 48)
    if nc % 48:
        sizes.append(nc % 48)
    return sizes


def _sc_segsum_body(h_hbm, src_hbm, dst_hbm, zeros_hbm, out_hbm,
                    src_v, dst_v, rows_v, acc_sh, gsems, ssems):
    c = lax.axis_index("c")
    s = lax.axis_index("s")

    lo = s * ROWS_MAIN

    @pl.when(c != 0)
    def _():
        pltpu.sync_copy(h_hbm.at[pl.ds(lo, ROWS_MAIN)],
                        acc_sh.at[pl.ds(lo, ROWS_MAIN)])

        @pl.when(s == 0)
        def _():
            pltpu.sync_copy(h_hbm.at[pl.ds(NS * ROWS_MAIN, TAIL)],
                            acc_sh.at[pl.ds(NS * ROWS_MAIN, TAIL)])

    @pl.when(c == 0)
    def _():
        pltpu.sync_copy(zeros_hbm.at[pl.ds(lo, ROWS_MAIN)],
                        acc_sh.at[pl.ds(lo, ROWS_MAIN)])

        @pl.when(s == 0)
        def _():
            pltpu.sync_copy(zeros_hbm.at[pl.ds(NS * ROWS_MAIN, TAIL)],
                            acc_sh.at[pl.ds(NS * ROWS_MAIN, TAIL)])

    plsc.subcore_barrier()

    def g_start(k, b):
        pltpu.async_copy(h_hbm.at[src_v.at[k]], rows_v.at[b], gsems[b])

    def g_wait(b):
        pltpu.make_async_copy(h_hbm.at[src_v.at[0]], rows_v.at[b],
                              gsems[b]).wait()

    def s_start(k, b):
        pltpu.async_copy(rows_v.at[b], acc_sh.at[dst_v.at[k]], ssems[b],
                         add=True)

    def s_wait(b):
        pltpu.make_async_copy(rows_v.at[b], acc_sh.at[dst_v.at[0]],
                              ssems[b]).wait()

    def run_edges(nc, base_row):
        off = 0
        for ph in _phase_sizes(nc):
            row0 = base_row + off
            off += ph
            pltpu.sync_copy(src_hbm.at[pl.ds(row0, ph)],
                            src_v.at[pl.ds(0, ph)])
            pltpu.sync_copy(dst_hbm.at[pl.ds(row0, ph)],
                            dst_v.at[pl.ds(0, ph)])
            g_start(0, 0)
            g_start(1, 1)

            @pl.loop(0, ph - 2, step=2)
            def _(c0):
                for b in range(2):
                    g_wait(b)
                    s_start(c0 + b, b)
                    s_wait(b)
                    g_start(c0 + b + 2, b)

            for k in (ph - 2, ph - 1):
                b = k % 2
                g_wait(b)
                s_start(k, b)
                s_wait(b)

    @pl.when(c == 0)
    def _():
        run_edges(NCH0, s * NCH0)

    if NCH1:
        @pl.when(c != 0)
        def _():
            run_edges(NCH1, NS * NCH0 + s * NCH1)

    plsc.subcore_barrier()

    pltpu.sync_copy(acc_sh.at[pl.ds(lo, ROWS_MAIN)],
                    out_hbm.at[c].at[pl.ds(lo, ROWS_MAIN)])

    @pl.when(s == 0)
    def _():
        pltpu.sync_copy(acc_sh.at[pl.ds(NS * ROWS_MAIN, TAIL)],
                        out_hbm.at[c].at[pl.ds(NS * ROWS_MAIN, TAIL)])


def _make_sc_segsum():
    mesh = plsc.VectorSubcoreMesh(core_axis_name="c", subcore_axis_name="s")
    maxph = max(_phase_sizes(NCH0) + _phase_sizes(NCH1))
    return pl.kernel(
        _sc_segsum_body,
        out_type=jax.ShapeDtypeStruct((NC, NP, EMB), jnp.float32),
        mesh=mesh,
        scratch_types=[
            pltpu.VMEM((maxph, CHUNK), jnp.int32),
            pltpu.VMEM((maxph, CHUNK), jnp.int32),
            pltpu.VMEM((NBUF, CHUNK, EMB), jnp.float32),
            pltpu.VMEM_SHARED((NP, EMB), jnp.float32),
            [pltpu.SemaphoreType.DMA] * NBUF,
            [pltpu.SemaphoreType.DMA] * NBUF,
        ],
    )


def _dense1_body(x_ref, w_ref, b_ref, o_ref):
    out = (jnp.dot(x_ref[...], w_ref[...],
                   preferred_element_type=jnp.float32) + b_ref[...])
    o_ref[...] = jnp.concatenate(
        [out, jnp.zeros((NP - N, EMB), jnp.float32)], axis=0)


def _layer_body(p_ref, wa_ref, ba_ref, g_ref, be_ref, wb_ref, bb_ref,
                o_ref, *, final_relu):
    agg = lax.slice(p_ref[0] + p_ref[1], (0, 0), (N, EMB))
    z = (jnp.dot(agg, wa_ref[...], preferred_element_type=jnp.float32)
         + ba_ref[...])
    mu = jnp.mean(z, axis=0, keepdims=True)
    var = jnp.mean((z - mu) ** 2, axis=0, keepdims=True)
    z = (z - mu) * lax.rsqrt(var + EPS) * g_ref[...] + be_ref[...]
    z = jnp.maximum(z, 0.0)
    out = (jnp.dot(z, wb_ref[...], preferred_element_type=jnp.float32)
           + bb_ref[...])
    if final_relu:
        out = jnp.maximum(out, 0.0)
    if o_ref.shape[0] == NP:
        out = jnp.concatenate(
            [out, jnp.zeros((NP - N, EMB), jnp.float32)], axis=0)
    o_ref[...] = out


def _dense1(x, w, b):
    return pl.pallas_call(
        _dense1_body,
        out_shape=jax.ShapeDtypeStruct((NP, EMB), jnp.float32),
    )(x, w, b.reshape(1, -1))


def _layer(p, wa, ba, g, be, wb, bb, final_relu, padded_out):
    rows = NP if padded_out else N
    return pl.pallas_call(
        functools.partial(_layer_body, final_relu=final_relu),
        out_shape=jax.ShapeDtypeStruct((rows, EMB), jnp.float32),
    )(p, wa, ba.reshape(1, -1), g.reshape(1, -1), be.reshape(1, -1),
      wb, bb.reshape(1, -1))


def kernel(x, edge_index, edge_attr, W1, b1, Wa0, ba0, g0, be0, Wb0, bb0,
           Wa1, ba1, g1, be1, Wb1, bb1):
    e = edge_index.shape[1]
    epad = NS * (NCH0 + NCH1) * CHUNK
    assert e <= epad
    pad = epad - e

    src = edge_index[0].astype(jnp.int32)
    dst = edge_index[1].astype(jnp.int32)
    if pad:
        src = jnp.concatenate([src, jnp.full((pad,), N, jnp.int32)])
        dst = jnp.concatenate([dst, jnp.arange(pad, dtype=jnp.int32) % N])
    src = src.reshape(epad // CHUNK, CHUNK)
    dst = dst.reshape(epad // CHUNK, CHUNK)
    zeros = jnp.zeros((NP, EMB), jnp.float32)

    segsum = _make_sc_segsum()

    h = _dense1(x, W1, b1)
    p = segsum(h, src, dst, zeros)
    h = _layer(p, Wa0, ba0, g0, be0, Wb0, bb0, final_relu=True,
               padded_out=True)
    p = segsum(h, src, dst, zeros)
    return _layer(p, Wa1, ba1, g1, be1, Wb1, bb1, final_relu=False,
                  padded_out=False)


# speedup vs baseline: 1.6050x; 1.6050x over previous
import functools

import jax
import jax.numpy as jnp
from jax import lax
from jax.experimental import pallas as pl
from jax.experimental.pallas import tpu as pltpu
from jax.experimental.pallas import tpu_sc as plsc

N = 10000
EMB = 128
EPS = 1e-5

NC = 2
NS = 16
NW = NC * NS
CHUNK = 128
NBUF = 2
NP = N + 8
ROWS_MAIN = (NP // NS) // 8 * 8
TAIL = NP - NS * ROWS_MAIN


NCH0 = 152
NCH1 = 8


def _phase_sizes(nc):
    sizes = [48] * (nc // 48)
    if nc % 48:
        sizes.append(nc % 48)
    return sizes


def _sc_segsum_body(h_hbm, src_hbm, dst_hbm, zeros_hbm, out_hbm,
                    src_v, dst_v, rows_v, acc_sh, gsems, ssems):
    c = lax.axis_index("c")
    s = lax.axis_index("s")

    lo = s * ROWS_MAIN

    @pl.when(c != 0)
    def _():
        pltpu.sync_copy(h_hbm.at[pl.ds(lo, ROWS_MAIN)],
                        acc_sh.at[pl.ds(lo, ROWS_MAIN)])

        @pl.when(s == 0)
        def _():
            pltpu.sync_copy(h_hbm.at[pl.ds(NS * ROWS_MAIN, TAIL)],
                            acc_sh.at[pl.ds(NS * ROWS_MAIN, TAIL)])

    @pl.when(c == 0)
    def _():
        pltpu.sync_copy(zeros_hbm.at[pl.ds(lo, ROWS_MAIN)],
                        acc_sh.at[pl.ds(lo, ROWS_MAIN)])

        @pl.when(s == 0)
        def _():
            pltpu.sync_copy(zeros_hbm.at[pl.ds(NS * ROWS_MAIN, TAIL)],
                            acc_sh.at[pl.ds(NS * ROWS_MAIN, TAIL)])

    plsc.subcore_barrier()

    def g_start(k, b):
        pltpu.async_copy(h_hbm.at[src_v.at[k]], rows_v.at[b], gsems[b])

    def g_wait(b):
        pltpu.make_async_copy(h_hbm.at[src_v.at[0]], rows_v.at[b],
                              gsems[b]).wait()

    def s_start(k, b):
        pltpu.async_copy(rows_v.at[b], acc_sh.at[dst_v.at[k]], ssems[b],
                         add=True)

    def s_wait(b):
        pltpu.make_async_copy(rows_v.at[b], acc_sh.at[dst_v.at[0]],
                              ssems[b]).wait()

    def run_edges(nc, base_row):
        off = 0
        for ph in _phase_sizes(nc):
            row0 = base_row + off
            off += ph
            pltpu.sync_copy(src_hbm.at[pl.ds(row0, ph)],
                            src_v.at[pl.ds(0, ph)])
            pltpu.sync_copy(dst_hbm.at[pl.ds(row0, ph)],
                            dst_v.at[pl.ds(0, ph)])
            g_start(0, 0)
            g_start(1, 1)

            @pl.loop(0, ph - 2, step=2)
            def _(c0):
                for b in range(2):
                    g_wait(b)
                    s_start(c0 + b, b)
                    s_wait(b)
                    g_start(c0 + b + 2, b)

            for k in (ph - 2, ph - 1):
                b = k % 2
                g_wait(b)
                s_start(k, b)
                s_wait(b)

    @pl.when(c == 0)
    def _():
        run_edges(NCH0, s * NCH0)

    if NCH1:
        @pl.when(c != 0)
        def _():
            run_edges(NCH1, NS * NCH0 + s * NCH1)

    plsc.subcore_barrier()

    pltpu.sync_copy(acc_sh.at[pl.ds(lo, ROWS_MAIN)],
                    out_hbm.at[c].at[pl.ds(lo, ROWS_MAIN)])

    @pl.when(s == 0)
    def _():
        pltpu.sync_copy(acc_sh.at[pl.ds(NS * ROWS_MAIN, TAIL)],
                        out_hbm.at[c].at[pl.ds(NS * ROWS_MAIN, TAIL)])


def _make_sc_segsum():
    mesh = plsc.VectorSubcoreMesh(core_axis_name="c", subcore_axis_name="s")
    maxph = max(_phase_sizes(NCH0) + _phase_sizes(NCH1))
    return pl.kernel(
        _sc_segsum_body,
        out_type=jax.ShapeDtypeStruct((NC, NP, EMB), jnp.float32),
        mesh=mesh,
        scratch_types=[
            pltpu.VMEM((maxph, CHUNK), jnp.int32),
            pltpu.VMEM((maxph, CHUNK), jnp.int32),
            pltpu.VMEM((NBUF, CHUNK, EMB), jnp.float32),
            pltpu.VMEM_SHARED((NP, EMB), jnp.float32),
            [pltpu.SemaphoreType.DMA] * NBUF,
            [pltpu.SemaphoreType.DMA] * NBUF,
        ],
    )


def _dense1_body(x_ref, w_ref, b_ref, o_ref):
    out = (jnp.dot(x_ref[...], w_ref[...],
                   preferred_element_type=jnp.float32) + b_ref[...])
    o_ref[...] = jnp.concatenate(
        [out, jnp.zeros((NP - N, EMB), jnp.float32)], axis=0)


def _layer_body(p_ref, wa_ref, ba_ref, g_ref, be_ref, wb_ref, bb_ref,
                o_ref, *, final_relu):
    agg = lax.slice(p_ref[0] + p_ref[1], (0, 0), (N, EMB))
    z = (jnp.dot(agg, wa_ref[...], preferred_element_type=jnp.float32)
         + ba_ref[...])
    mu = jnp.mean(z, axis=0, keepdims=True)
    var = jnp.mean((z - mu) ** 2, axis=0, keepdims=True)
    z = (z - mu) * lax.rsqrt(var + EPS) * g_ref[...] + be_ref[...]
    z = jnp.maximum(z, 0.0)
    out = (jnp.dot(z, wb_ref[...], preferred_element_type=jnp.float32)
           + bb_ref[...])
    if final_relu:
        out = jnp.maximum(out, 0.0)
    if o_ref.shape[0] == NP:
        out = jnp.concatenate(
            [out, jnp.zeros((NP - N, EMB), jnp.float32)], axis=0)
    o_ref[...] = out


def _dense1(x, w, b):
    return pl.pallas_call(
        _dense1_body,
        out_shape=jax.ShapeDtypeStruct((NP, EMB), jnp.float32),
    )(x, w, b.reshape(1, -1))


def _layer(p, wa, ba, g, be, wb, bb, final_relu, padded_out):
    rows = NP if padded_out else N
    return pl.pallas_call(
        functools.partial(_layer_body, final_relu=final_relu),
        out_shape=jax.ShapeDtypeStruct((rows, EMB), jnp.float32),
    )(p, wa, ba.reshape(1, -1), g.reshape(1, -1), be.reshape(1, -1),
      wb, bb.reshape(1, -1))


def kernel(x, edge_index, edge_attr, W1, b1, Wa0, ba0, g0, be0, Wb0, bb0,
           Wa1, ba1, g1, be1, Wb1, bb1):
    e = edge_index.shape[1]
    epad = NS * (NCH0 + NCH1) * CHUNK
    assert e <= epad
    pad = epad - e

    src = edge_index[0].astype(jnp.int32)
    dst = edge_index[1].astype(jnp.int32)
    if pad:
        src = jnp.concatenate([src, jnp.full((pad,), N, jnp.int32)])
        dst = jnp.concatenate([dst, jnp.arange(pad, dtype=jnp.int32) % N])
    src = src.reshape(epad // CHUNK, CHUNK)
    dst = dst.reshape(epad // CHUNK, CHUNK)
    zeros = jnp.zeros((NP, EMB), jnp.float32)

    segsum = _make_sc_segsum()

    h = _dense1(x, W1, b1)
    p = segsum(h, src, dst, zeros)
    h = _layer(p, Wa0, ba0, g0, be0, Wb0, bb0, final_relu=True,
               padded_out=True)
    p = segsum(h, src, dst, zeros)
    return _layer(p, Wa1, ba1, g1, be1, Wb1, bb1, final_relu=False,
                  padded_out=False)
